# baseline (speedup 1.0000x reference)
import jax
import jax.numpy as jnp
import numpy as np
from jax.experimental import pallas as pl

_FREQS = tuple(float(f) for f in (100.0 ** (np.arange(8) / 8.0)).astype(np.float32))
_SCALES = (256, 128, 64, 32, 16, 8)
_CHANNELS = (64, 64, 128, 256, 512, 1024)

_K = np.arange(5, dtype=np.float64) - 2.0
_PDF = np.exp(-0.5 * _K**2)
_K1 = (_PDF / _PDF.sum()).astype(np.float32)
_POLY = tuple(float(v) for v in np.polyfit(np.arange(5.0), _K1.astype(np.float64), 4))


def _np_col(S):
    r = np.arange(S)[:, None]
    y = np.arange(S)[None, :]
    k1 = _K1.astype(np.float64)

    def tap(t, gate):
        w = np.zeros_like(t, dtype=np.float64)
        m = (t >= 0) & (t <= 4) & gate
        w[m] = k1[t[m]]
        return w

    c = tap(r - y + 2, np.ones_like(r - y, bool))
    c += tap(2 - r - y, r >= 1)
    c += tap(2 * S - r - y, r <= S - 2)
    return c


def _np_blur(S):
    c = _np_col(S)
    b = np.einsum('ry,dx->rdyx', c, c).reshape(S * S, S * S)
    return jnp.asarray(b.astype(np.float32)).astype(jnp.bfloat16)


_DENSE_BLUR_SCALES = (32, 16, 8)


def _tapw(t, extra_gate=None):
    tf = t.astype(jnp.float32)
    w = _POLY[0]
    for c in _POLY[1:]:
        w = w * tf + c
    m = (t >= 0) & (t <= 4)
    if extra_gate is not None:
        m = m & extra_gate
    return jnp.where(m, w, 0.0)


def _footprint(idx, pos, S):
    w = _tapw(idx - pos + 2)
    w = w + _tapw(2 - idx - pos, idx >= 1)
    w = w + _tapw(2 * S - idx - pos, idx <= S - 2)
    return w


def _mlp_T(wslice, fT, ones):
    w1, b1, w2, b2, w3, b3 = wslice
    dn = (((0,), (0,)), ((), ()))
    w1v = w1[...].reshape(8, 4, -1)
    w1a = jnp.concatenate(
        [w1v[:, 0, :], w1v[:, 1, :], w1v[:, 2, :], w1v[:, 3, :],
         b1[...].reshape(1, -1)], axis=0)
    h = jax.lax.dot_general(w1a, fT, dn, preferred_element_type=jnp.float32)
    h = h * jax.nn.sigmoid(h)
    h = jnp.concatenate([h, ones], axis=0)
    w2a = jnp.concatenate([w2[...], b2[...].reshape(1, -1)], axis=0)
    h = jax.lax.dot_general(w2a, h, dn, preferred_element_type=jnp.float32)
    h = h * jax.nn.sigmoid(h)
    h = jnp.concatenate([h, ones], axis=0)
    w3a = jnp.concatenate([w3[...], b3[...].reshape(1, -1)], axis=0)
    return jax.lax.dot_general(w3a, h, dn, preferred_element_type=jnp.float32)


_YSPLIT = 4


def _body(ct_ref, cn_ref, fcol_ref, *refs, scales, channels):
    nw = sum(6 + (1 if S in _DENSE_BLUR_SCALES else 0) for S in scales)
    wrefs = refs[:nw]
    outs = refs[nw:]
    N = ct_ref.shape[2]
    j = pl.program_id(1)

    ct = ct_ref[0]
    x0 = jnp.concatenate([ct[0:1, :], ct[2:3, :]], axis=1)
    x1 = jnp.concatenate([ct[1:2, :], ct[3:4, :]], axis=1)

    fcol = fcol_ref[...]
    ones = jnp.full((1, 2 * N), 1.0, jnp.float32)
    fT = jnp.concatenate(
        [jnp.sin(fcol * x0), jnp.sin(fcol * x1),
         jnp.cos(fcol * x0), jnp.cos(fcol * x1), ones],
        axis=0)

    cn = cn_ref[0]

    S0 = scales[0]
    c20 = channels[0] // 2
    SB = S0 // _YSPLIT
    inv0 = 1.0 / (512 // S0)
    eT0 = _mlp_T(wrefs[0:6], fT, ones)
    yi0 = jax.lax.broadcasted_iota(jnp.int32, (SB, N), 0) + j * SB
    xi0 = jax.lax.broadcasted_iota(jnp.int32, (N, S0), 1)
    for half in range(2):
        eh = eT0[:, half * N:(half + 1) * N].astype(jnp.bfloat16)
        r_row = (ct[2 * half:2 * half + 1, :] * inv0).astype(jnp.int32)
        c_col = (cn[:, 2 * half + 1:2 * half + 2] * inv0).astype(jnp.int32)
        wy = _footprint(r_row, yi0, S0).astype(jnp.bfloat16)
        wx = _footprint(c_col, xi0, S0).astype(jnp.bfloat16)
        p = (eh[:, None, :] * wy[None, :, :]).reshape(c20 * SB, N)
        out = jnp.dot(p, wx, preferred_element_type=jnp.float32)
        outs[half][...] = out.reshape(1, c20, SB, S0)

    @pl.when(j == 0)
    def _rest():
        woff = 6
        for i in range(1, len(scales)):
            S = scales[i]
            c2 = channels[i] // 2
            dense = S in _DENSE_BLUR_SCALES
            wslice = wrefs[woff:woff + 6]
            blur_ref = wrefs[woff + 6] if dense else None
            eT = _mlp_T(wslice, fT, ones)

            ratio = 512 // S
            inv = 1.0 / ratio
            if dense:
                li = jax.lax.broadcasted_iota(jnp.int32, (N, S * S), 1)
            else:
                yi = jax.lax.broadcasted_iota(jnp.int32, (S, N), 0)
                xi = jax.lax.broadcasted_iota(jnp.int32, (N, S), 1)
            for half in range(2):
                eh = eT[:, half * N:(half + 1) * N].astype(jnp.bfloat16)
                if dense:
                    r_col = (cn[:, 2 * half:2 * half + 1] * inv).astype(jnp.int32)
                    c_col = (cn[:, 2 * half + 1:2 * half + 2] * inv).astype(jnp.int32)
                    cell = r_col * S + c_col
                    oh = jnp.where(li == cell, 1.0, 0.0).astype(jnp.bfloat16)
                    g = jnp.dot(eh, oh, preferred_element_type=jnp.float32)
                    out = jnp.dot(g.astype(jnp.bfloat16), blur_ref[...],
                                  preferred_element_type=jnp.float32)
                else:
                    r_row = (ct[2 * half:2 * half + 1, :] * inv).astype(jnp.int32)
                    c_col = (cn[:, 2 * half + 1:2 * half + 2] * inv).astype(jnp.int32)
                    wy = _footprint(r_row, yi, S).astype(jnp.bfloat16)
                    wx = _footprint(c_col, xi, S).astype(jnp.bfloat16)
                    p = (eh[:, None, :] * wy[None, :, :]).reshape(c2 * S, N)
                    out = jnp.dot(p, wx, preferred_element_type=jnp.float32)
                outs[2 * i + half][...] = out.reshape(1, c2, S, S)
            woff += 7 if dense else 6


def _run_group(ct, cn, params, scales, channels, BV, N):
    import functools
    body = functools.partial(_body, scales=scales, channels=channels)
    weight_args, weight_specs = [], []
    out_shapes, out_specs = [], []
    full = lambda a: pl.BlockSpec(a.shape, lambda i, j: (0,) * a.ndim)
    for i, S in enumerate(scales):
        c2 = channels[i] // 2
        p = params[i]
        args = [p['W1'], p['b1'], p['W2'], p['b2'], p['W3'], p['b3']]
        if S in _DENSE_BLUR_SCALES:
            args.append(_np_blur(S))
        for a in args:
            weight_args.append(a)
            weight_specs.append(full(a))
        for _ in range(2):
            out_shapes.append(jax.ShapeDtypeStruct((BV, c2, S, S), jnp.float32))
            if i == 0:
                out_specs.append(pl.BlockSpec(
                    (1, c2, S // _YSPLIT, S), lambda i, j: (i, 0, j, 0)))
            else:
                out_specs.append(pl.BlockSpec(
                    (1, c2, S, S), lambda i, j: (i, 0, 0, 0)))

    return pl.pallas_call(
        body,
        grid=(BV, _YSPLIT),
        in_specs=[
            pl.BlockSpec((1, 4, N), lambda i, j: (i, 0, 0)),
            pl.BlockSpec((1, N, 4), lambda i, j: (i, 0, 0)),
            pl.BlockSpec((8, 1), lambda i, j: (0, 0)),
            *weight_specs,
        ],
        out_specs=out_specs,
        out_shape=out_shapes,
    )(ct, cn, jnp.asarray(np.asarray(_FREQS, np.float32)[:, None]), *weight_args)


def kernel(drags_start, drags_end, params):
    B, V, N, _ = drags_start.shape
    BV = B * V
    ds = drags_start.reshape(BV, N, 2)
    de = drags_end.reshape(BV, N, 2)
    cn = jnp.concatenate([ds, de], axis=2)
    ct = jnp.transpose(cn, (0, 2, 1))

    outs = list(_run_group(ct, cn, params, _SCALES, _CHANNELS, BV, N))

    outs_s = [outs[2 * i] for i in range(len(_SCALES))]
    outs_e = [outs[2 * i + 1] for i in range(len(_SCALES))]
    return (outs_s, outs_e)


# speedup vs baseline: 1.1398x; 1.1398x over previous
import jax
import jax.numpy as jnp
import numpy as np
from jax.experimental import pallas as pl

_FREQS = tuple(float(f) for f in (100.0 ** (np.arange(8) / 8.0)).astype(np.float32))
_SCALES = (256, 128, 64, 32, 16, 8)
_CHANNELS = (64, 64, 128, 256, 512, 1024)

_K = np.arange(5, dtype=np.float64) - 2.0
_PDF = np.exp(-0.5 * _K**2)
_K1 = (_PDF / _PDF.sum()).astype(np.float32)
_POLY = tuple(float(v) for v in np.polyfit(np.arange(5.0), _K1.astype(np.float64), 4))


def _np_col(S):
    r = np.arange(S)[:, None]
    y = np.arange(S)[None, :]
    k1 = _K1.astype(np.float64)

    def tap(t, gate):
        w = np.zeros_like(t, dtype=np.float64)
        m = (t >= 0) & (t <= 4) & gate
        w[m] = k1[t[m]]
        return w

    c = tap(r - y + 2, np.ones_like(r - y, bool))
    c += tap(2 - r - y, r >= 1)
    c += tap(2 * S - r - y, r <= S - 2)
    return c


def _np_blur(S):
    c = _np_col(S)
    b = np.einsum('ry,dx->rdyx', c, c).reshape(S * S, S * S)
    return jnp.asarray(b.astype(np.float32)).astype(jnp.bfloat16)


_DENSE_BLUR_SCALES = (32, 16, 8)


def _tapw(t, extra_gate=None):
    tf = t.astype(jnp.float32)
    w = _POLY[0]
    for c in _POLY[1:]:
        w = w * tf + c
    m = (t >= 0) & (t <= 4)
    if extra_gate is not None:
        m = m & extra_gate
    return jnp.where(m, w, 0.0)


def _footprint(idx, pos, S):
    w = _tapw(idx - pos + 2)
    w = w + _tapw(2 - idx - pos, idx >= 1)
    w = w + _tapw(2 * S - idx - pos, idx <= S - 2)
    return w


def _footprint_T(idx, pos, S):
    w = _tapw(idx - pos + 2)
    top = w[0:2] + _tapw(2 - idx - pos[0:2], idx >= 1)
    bot = w[S - 2:] + _tapw(2 * S - idx - pos[S - 2:], idx <= S - 2)
    return jnp.concatenate([top, w[2:S - 2], bot], axis=0)


def _mlp_T(wslice, fT, ones):
    w1, b1, w2, b2, w3, b3 = wslice
    dn = (((0,), (0,)), ((), ()))
    w1v = w1[...].reshape(8, 4, -1)
    w1a = jnp.concatenate(
        [w1v[:, 0, :], w1v[:, 1, :], w1v[:, 2, :], w1v[:, 3, :],
         b1[...].reshape(1, -1)], axis=0)
    h = jax.lax.dot_general(w1a, fT, dn, preferred_element_type=jnp.float32)
    h = h * jax.nn.sigmoid(h)
    h = jnp.concatenate([h, ones], axis=0)
    w2a = jnp.concatenate([w2[...], b2[...].reshape(1, -1)], axis=0)
    h = jax.lax.dot_general(w2a, h, dn, preferred_element_type=jnp.float32)
    h = h * jax.nn.sigmoid(h)
    h = jnp.concatenate([h, ones], axis=0)
    w3a = jnp.concatenate([w3[...], b3[...].reshape(1, -1)], axis=0)
    return jax.lax.dot_general(w3a, h, dn, preferred_element_type=jnp.float32)


def _body(ct_ref, cn_ref, fcol_ref, *refs, scales, channels):
    nw = sum(6 + (1 if S in _DENSE_BLUR_SCALES else 0) for S in scales)
    wrefs = refs[:nw]
    outs = refs[nw:]
    N = ct_ref.shape[2]

    ct = ct_ref[0]
    x0 = jnp.concatenate([ct[0:1, :], ct[2:3, :]], axis=1)
    x1 = jnp.concatenate([ct[1:2, :], ct[3:4, :]], axis=1)

    fcol = fcol_ref[...]
    ones = jnp.full((1, 2 * N), 1.0, jnp.float32)
    fT = jnp.concatenate(
        [jnp.sin(fcol * x0), jnp.sin(fcol * x1),
         jnp.cos(fcol * x0), jnp.cos(fcol * x1), ones],
        axis=0)

    cn = cn_ref[0]

    woff = 0
    for i, S in enumerate(scales):
        c2 = channels[i] // 2
        dense = S in _DENSE_BLUR_SCALES
        wslice = wrefs[woff:woff + 6]
        blur_ref = wrefs[woff + 6] if dense else None
        woff += 7 if dense else 6
        eT = _mlp_T(wslice, fT, ones)

        ratio = 512 // S
        inv = 1.0 / ratio
        if dense:
            li = jax.lax.broadcasted_iota(jnp.int32, (N, S * S), 1)
        else:
            yi = jax.lax.broadcasted_iota(jnp.int32, (S, N), 0)
            xi = jax.lax.broadcasted_iota(jnp.int32, (N, S), 1)
        for half in range(2):
            eh = eT[:, half * N:(half + 1) * N].astype(jnp.bfloat16)
            if dense:
                r_col = (cn[:, 2 * half:2 * half + 1] * inv).astype(jnp.int32)
                c_col = (cn[:, 2 * half + 1:2 * half + 2] * inv).astype(jnp.int32)
                cell = r_col * S + c_col
                oh = jnp.where(li == cell, 1.0, 0.0).astype(jnp.bfloat16)
                g = jnp.dot(eh, oh, preferred_element_type=jnp.float32)
                out = jnp.dot(g.astype(jnp.bfloat16), blur_ref[...],
                              preferred_element_type=jnp.float32)
            else:
                r_row = (ct[2 * half:2 * half + 1, :] * inv).astype(jnp.int32)
                c_col = (cn[:, 2 * half + 1:2 * half + 2] * inv).astype(jnp.int32)
                wy = _footprint_T(r_row, yi, S).astype(jnp.bfloat16)
                wx = _footprint(c_col, xi, S).astype(jnp.bfloat16)
                p = (eh[:, None, :] * wy[None, :, :]).reshape(c2 * S, N)
                out = jnp.dot(p, wx, preferred_element_type=jnp.float32)
            outs[2 * i + half][...] = out.reshape(1, c2, S, S)


def _run_group(ct, cn, params, scales, channels, BV, N):
    import functools
    body = functools.partial(_body, scales=scales, channels=channels)
    weight_args, weight_specs = [], []
    out_shapes, out_specs = [], []
    full = lambda a: pl.BlockSpec(a.shape, lambda i: (0,) * a.ndim)
    for i, S in enumerate(scales):
        c2 = channels[i] // 2
        p = params[i]
        args = [p['W1'], p['b1'], p['W2'], p['b2'], p['W3'], p['b3']]
        if S in _DENSE_BLUR_SCALES:
            args.append(_np_blur(S))
        for a in args:
            weight_args.append(a)
            weight_specs.append(full(a))
        for _ in range(2):
            out_shapes.append(jax.ShapeDtypeStruct((BV, c2, S, S), jnp.float32))
            out_specs.append(pl.BlockSpec((1, c2, S, S), lambda i: (i, 0, 0, 0)))

    return pl.pallas_call(
        body,
        grid=(BV,),
        in_specs=[
            pl.BlockSpec((1, 4, N), lambda i: (i, 0, 0)),
            pl.BlockSpec((1, N, 4), lambda i: (i, 0, 0)),
            pl.BlockSpec((8, 1), lambda i: (0, 0)),
            *weight_specs,
        ],
        out_specs=out_specs,
        out_shape=out_shapes,
    )(ct, cn, jnp.asarray(np.asarray(_FREQS, np.float32)[:, None]), *weight_args)


def kernel(drags_start, drags_end, params):
    B, V, N, _ = drags_start.shape
    BV = B * V
    ds = drags_start.reshape(BV, N, 2)
    de = drags_end.reshape(BV, N, 2)
    cn = jnp.concatenate([ds, de], axis=2)
    ct = jnp.transpose(cn, (0, 2, 1))

    outs_a = _run_group(ct, cn, params[:1], _SCALES[:1], _CHANNELS[:1], BV, N)
    outs_b = _run_group(ct, cn, params[1:], _SCALES[1:], _CHANNELS[1:], BV, N)
    outs = list(outs_a) + list(outs_b)

    outs_s = [outs[2 * i] for i in range(len(_SCALES))]
    outs_e = [outs[2 * i + 1] for i in range(len(_SCALES))]
    return (outs_s, outs_e)
